# baseline (speedup 1.0000x reference)
import functools

import jax
import jax.numpy as jnp
from jax import lax
from jax.experimental import pallas as pl
from jax.experimental.pallas import tpu as pltpu
from jax.experimental.pallas import tpu_sc as plsc

NUM_EMB = 1_000_000
DIM = 64
BATCH = 16384
NS, L = 16, 16
NW = NS
RANGE = 65536
POS_PAD = RANGE * NW
B_PER_W = BATCH // NW
CHUNK = 128
NCHUNK = B_PER_W // CHUNK

_mesh = plsc.VectorSubcoreMesh(
    core_axis_name="c", subcore_axis_name="s", num_cores=1)


def _lane_shift_up(x, lanes):
    idx = jnp.minimum(lanes + 1, L - 1)
    dnums = lax.GatherDimensionNumbers(
        offset_dims=(), collapsed_slice_dims=(0,), start_index_map=(0,))
    return lax.gather(x, idx[:, None], dnums, slice_sizes=(1,),
                      mode=lax.GatherScatterMode.PROMISE_IN_BOUNDS)


@functools.partial(
    pl.kernel,
    mesh=_mesh,
    compiler_params=pltpu.CompilerParams(needs_layout_passes=False),
    out_type=jax.ShapeDtypeStruct((POS_PAD,), jnp.int32),
    scratch_types=[
        pltpu.VMEM((BATCH,), jnp.int32),
        pltpu.VMEM((RANGE,), jnp.int32),
    ],
)
def _build_pos(nid_hbm, pos_hbm, nid_v, slab_v):
    wid = lax.axis_index("s")
    pltpu.sync_copy(nid_hbm, nid_v)
    lanes = lax.iota(jnp.int32, L)

    def body(it, carry):
        v = nid_v[pl.ds(it * L, L)]
        key = (v << 4) | lanes
        skey, _ = plsc.sort_key_val(key, key)
        sv = lax.shift_right_arithmetic(skey, 4)
        nxt = _lane_shift_up(sv, lanes)
        keep = (sv != nxt) | (lanes == L - 1)
        m = keep & (lax.shift_right_arithmetic(sv, 16) == wid)
        j = it * L + (skey & (L - 1))
        plsc.store_scatter(slab_v, [sv & (RANGE - 1)], j, mask=m)
        return carry

    lax.fori_loop(0, BATCH // L, body, 0, unroll=4)
    pltpu.sync_copy(slab_v, pos_hbm.at[pl.ds(wid * RANGE, RANGE)])


@functools.partial(
    pl.kernel,
    mesh=_mesh,
    compiler_params=pltpu.CompilerParams(
        needs_layout_passes=False, use_tc_tiling_on_sc=False),
    out_type=jax.ShapeDtypeStruct((NW, B_PER_W, DIM), jnp.float32),
    scratch_types=[
        pltpu.VMEM((NCHUNK, CHUNK), jnp.int32),
        pltpu.VMEM((NCHUNK, CHUNK), jnp.int32),
        pltpu.VMEM((B_PER_W, DIM), jnp.float32),
        pltpu.SemaphoreType.DMA,
    ],
)
def _gather_out(nid_hbm, pos_hbm, x_hbm, out_hbm, nid_v, w_v, rows_v, sem):
    wid = lax.axis_index("s")
    pltpu.sync_copy(nid_hbm.at[pl.ds(wid * NCHUNK, NCHUNK)], nid_v)
    h1 = [pltpu.async_copy(pos_hbm.at[nid_v.at[c]], w_v.at[c], sem)
          for c in range(NCHUNK)]
    for h in h1:
        h.wait()
    h2 = [pltpu.async_copy(x_hbm.at[w_v.at[c]],
                           rows_v.at[pl.ds(c * CHUNK, CHUNK)], sem)
          for c in range(NCHUNK)]
    for h in h2:
        h.wait()
    pltpu.sync_copy(rows_v, out_hbm.at[wid])


def kernel(emb, x, n_id):
    del emb
    pos = _build_pos(n_id)
    nid2 = n_id.reshape(NW * NCHUNK, CHUNK)
    out = _gather_out(nid2, pos, x)
    return out.reshape(BATCH, DIM)


# speedup vs baseline: 1.0768x; 1.0768x over previous
import functools

import jax
import jax.numpy as jnp
from jax import lax
from jax.experimental import pallas as pl
from jax.experimental.pallas import tpu as pltpu
from jax.experimental.pallas import tpu_sc as plsc

NUM_EMB = 1_000_000
DIM = 64
BATCH = 16384
NS, L = 16, 16
NW = NS
RANGE = 65536
POS_PAD = RANGE * NW
B_PER_W = BATCH // NW
CHUNK = 128
NCHUNK = B_PER_W // CHUNK

_mesh = plsc.VectorSubcoreMesh(
    core_axis_name="c", subcore_axis_name="s", num_cores=1)


@functools.partial(
    pl.kernel,
    mesh=_mesh,
    compiler_params=pltpu.CompilerParams(needs_layout_passes=False),
    out_type=jax.ShapeDtypeStruct((POS_PAD,), jnp.int32),
    scratch_types=[
        pltpu.VMEM((BATCH,), jnp.int32),
        pltpu.VMEM((RANGE,), jnp.int32),
    ],
)
def _build_pos(nid_hbm, pos_hbm, nid_v, slab_v):
    wid = lax.axis_index("s")
    pltpu.sync_copy(nid_hbm, nid_v)
    lanes = lax.iota(jnp.int32, L)

    def body(it, carry):
        v = nid_v[pl.ds(it * L, L)]
        _, last = plsc.scan_count(v)
        m = last & (lax.shift_right_arithmetic(v, 16) == wid)
        j = it * L + lanes
        plsc.store_scatter(slab_v, [v & (RANGE - 1)], j, mask=m)
        return carry

    lax.fori_loop(0, BATCH // L, body, 0, unroll=4)
    pltpu.sync_copy(slab_v, pos_hbm.at[pl.ds(wid * RANGE, RANGE)])


@functools.partial(
    pl.kernel,
    mesh=_mesh,
    compiler_params=pltpu.CompilerParams(
        needs_layout_passes=False, use_tc_tiling_on_sc=False),
    out_type=jax.ShapeDtypeStruct((NW, B_PER_W, DIM), jnp.float32),
    scratch_types=[
        pltpu.VMEM((NCHUNK, CHUNK), jnp.int32),
        pltpu.VMEM((NCHUNK, CHUNK), jnp.int32),
        pltpu.VMEM((B_PER_W, DIM), jnp.float32),
        pltpu.SemaphoreType.DMA,
    ],
)
def _gather_out(nid_hbm, pos_hbm, x_hbm, out_hbm, nid_v, w_v, rows_v, sem):
    wid = lax.axis_index("s")
    pltpu.sync_copy(nid_hbm.at[pl.ds(wid * NCHUNK, NCHUNK)], nid_v)
    h1 = [pltpu.async_copy(pos_hbm.at[nid_v.at[c]], w_v.at[c], sem)
          for c in range(NCHUNK)]
    for h in h1:
        h.wait()
    h2 = [pltpu.async_copy(x_hbm.at[w_v.at[c]],
                           rows_v.at[pl.ds(c * CHUNK, CHUNK)], sem)
          for c in range(NCHUNK)]
    for h in h2:
        h.wait()
    pltpu.sync_copy(rows_v, out_hbm.at[wid])


def kernel(emb, x, n_id):
    del emb
    pos = _build_pos(n_id)
    nid2 = n_id.reshape(NW * NCHUNK, CHUNK)
    out = _gather_out(nid2, pos, x)
    return out.reshape(BATCH, DIM)
